# baseline (speedup 1.0000x reference)
import functools

import jax
import jax.numpy as jnp
from jax import lax
from jax.experimental import pallas as pl
from jax.experimental.pallas import tpu as pltpu
from jax.experimental.pallas import tpu_sc as plsc

T = 2048; HID = 2048; H = 16; QLORA = 1536; KVLORA = 512
ROPE = 64; NOPE = 128; QKHD = 192; VHD = 128
IH = 16; IHD = 128; TOPK = 128
SCALE = QKHD ** -0.5
WSCALE = (IHD ** -0.5) * (IH ** -0.5)
NEG_INF = float("-inf")

HIGH = "bf16"
DEFAULT = "bf16"

C_QL = 0
C_KC = 1536
C_KI = 2048
C_KR = 2176
C_W = 2304
K1_COLS = 2432

K2_COLS = 5120


def _dot(a, b, prec):
    if prec == "bf16":
        a = a.astype(jnp.bfloat16)
        b = b.astype(jnp.bfloat16)
        prec = None
    return jax.lax.dot_general(a, b, (((1,), (0,)), ((), ())),
                               precision=prec, preferred_element_type=jnp.float32)


def _dot_bT(a, b, prec):
    if prec == "bf16":
        a = a.astype(jnp.bfloat16)
        b = b.astype(jnp.bfloat16)
        prec = None
    return jax.lax.dot_general(a, b, (((1,), (1,)), ((), ())),
                               precision=prec, preferred_element_type=jnp.float32)


def _rms(x, w, eps=1e-6):
    return x * jax.lax.rsqrt(jnp.mean(x * x, -1, keepdims=True) + eps) * w


def _ln(x, w, b, eps=1e-6):
    mu = jnp.mean(x, -1, keepdims=True)
    var = jnp.mean((x - mu) ** 2, -1, keepdims=True)
    return (x - mu) * jax.lax.rsqrt(var + eps) * w + b


def _swap_pairs(x):
    even = jax.lax.broadcasted_iota(jnp.int32, x.shape, x.ndim - 1) % 2 == 0
    return jnp.where(even, jnp.roll(x, -1, axis=-1), jnp.roll(x, 1, axis=-1))


def _rope_il(x, cos_il, sin_sgn):
    return x * cos_il + _swap_pairs(x) * sin_sgn


def _k1_body(x_ref, wqa_ref, wkva_ref, wik_ref, wiw_ref,
             qaln_ref, kvln_ref, ikw_ref, ikb_ref, cos_ref, sin_ref,
             qlat_ref, kv_ref, kvT_ref, kidxT_ref, wind_ref):
    x = x_ref[...]
    qlat_ref[...] = _rms(_dot_bT(x, wqa_ref[...], HIGH), qaln_ref[...])
    ckv = _dot_bT(x, wkva_ref[...], HIGH)
    k_comp = _rms(ckv[:, :KVLORA], kvln_ref[...])
    cos = cos_ref[...]; sin = sin_ref[...]
    k_rope = _rope_il(ckv[:, KVLORA:], cos, sin)
    kv = jnp.concatenate([k_comp, k_rope], axis=1)
    kv_ref[...] = kv
    kvT_ref[...] = kv.T
    ki = _ln(_dot_bT(x, wik_ref[...], HIGH), ikw_ref[...], ikb_ref[...])
    tb = ki.shape[0]
    ccat = jnp.concatenate([cos, jnp.ones((tb, IHD - ROPE), jnp.float32)], axis=1)
    scat = jnp.concatenate([sin, jnp.zeros((tb, IHD - ROPE), jnp.float32)], axis=1)
    kidxT_ref[...] = (ki * ccat + _swap_pairs(ki) * scat).T
    wind_ref[...] = _dot_bT(x, wiw_ref[...], HIGH)


def _k2a_body(qlat_ref, wiq_ref, cos_ref, sin_ref, qi_ref):
    ql = qlat_ref[...]
    cos = cos_ref[...]; sin = sin_ref[...]
    tb = ql.shape[0]
    ccat = jnp.concatenate([cos, jnp.ones((tb, IHD - ROPE), jnp.float32)], axis=1)
    scat = jnp.concatenate([sin, jnp.zeros((tb, IHD - ROPE), jnp.float32)], axis=1)
    for h in range(IH):
        qh = _dot_bT(ql, wiq_ref[h * IHD:(h + 1) * IHD], HIGH)
        qi_ref[:, h * IHD:(h + 1) * IHD] = qh * ccat + _swap_pairs(qh) * scat


def _k2b_body(qlat_ref, wqb_ref, wkvb_ref, cos_ref, sin_ref, sq_ref):
    ql = qlat_ref[...]
    cos = cos_ref[...]; sin = sin_ref[...]
    for h in range(H):
        qf = _dot_bT(ql, wqb_ref[h * QKHD:(h + 1) * QKHD], HIGH)
        qa = _dot(qf[:, :NOPE],
                  wkvb_ref[h * (NOPE + VHD):h * (NOPE + VHD) + NOPE], HIGH)
        qr = _rope_il(qf[:, NOPE:], cos, sin)
        sq_ref[h] = jnp.concatenate([qa, qr], axis=1)


def _k3_body(qi_ref, kT_ref, w_ref, ks_ref, ke_ref, out_ref):
    tb = qi_ref.shape[0]
    acc = jnp.zeros((tb, T), jnp.float32)
    w = w_ref[...]
    for h in range(IH):
        s = _dot(qi_ref[:, h * IHD:(h + 1) * IHD], kT_ref[...], HIGH)
        acc = acc + jnp.maximum(s, 0.0) * w[:, h:h + 1]
    acc = acc * WSCALE
    pos = jax.lax.broadcasted_iota(jnp.int32, (tb, T), 1)
    valid = (pos >= ks_ref[:, :1]) & (pos < ke_ref[:, :1])
    out_ref[...] = jnp.where(valid, acc, NEG_INF)


def _k4_body(sq_ref, kvT_ref, kv_ref, mask_ref, out_ref):
    q = sq_ref[0]
    l = _dot(q, kvT_ref[...], DEFAULT) * SCALE + mask_ref[...]
    m = jnp.max(l, axis=-1, keepdims=True)
    p = jnp.exp(l - m)
    p = p / jnp.sum(p, axis=-1, keepdims=True)
    out_ref[0] = _dot(p, kv_ref[:, :KVLORA], DEFAULT)


def _k5_body(ao_ref, wkvb_ref, wo_ref, out_ref):
    cat = jnp.concatenate(
        [_dot_bT(ao_ref[h],
                 wkvb_ref[h * (NOPE + VHD) + NOPE:(h + 1) * (NOPE + VHD)],
                 DEFAULT) for h in range(H)], axis=1)
    out_ref[...] = _dot_bT(cat, wo_ref[...], DEFAULT)


def _const_spec(shape):
    return pl.BlockSpec(shape, lambda *args: tuple(0 for _ in shape))


_NW = 32
_RPW = T // _NW
_NCH = T // 16
_MINF_UKEY = 0x007FFFFF


def _lanes():
    return lax.iota(jnp.int32, 16)


def _tree_sum(v):
    ln = _lanes()
    for sh in (1, 2, 4, 8):
        v = v + jnp.take(v, ln ^ sh)
    return v


def _prefix_incl(v):
    ln = _lanes()
    for sh in (1, 2, 4, 8):
        t = jnp.take(v, jnp.maximum(ln - sh, 0))
        v = v + jnp.where(ln >= sh, t, 0)
    return v


def _sc_topk_body(scores_hbm, mask_hbm, row_v, out_v, key_v):
    wid = lax.axis_index("s") * 2 + lax.axis_index("c")
    one_u = jnp.full((16,), 1, jnp.uint32)
    k128 = jnp.full((16,), TOPK, jnp.int32)
    base = wid * _RPW

    def process_group(g, carry):
        pltpu.sync_copy(scores_hbm.at[pl.ds(base + 8 * g, 8)], row_v)
        lax.fori_loop(0, 8, process_row, g)
        pltpu.sync_copy(out_v, mask_hbm.at[pl.ds(base + 8 * g, 8)])
        return carry

    def process_row(j, carry):
        def xform(c, _):
            for u4 in range(4):
                x = row_v[j, pl.ds(c * 64 + u4 * 16, 16)]
                u = lax.bitcast_convert_type(x, jnp.uint32)
                flip = jnp.where(u >> 31 != 0,
                                 jnp.uint32(0xFFFFFFFF), jnp.uint32(0x80000000))
                key_v[pl.ds(c * 64 + u4 * 16, 16)] = u ^ flip
            return 0
        lax.fori_loop(0, _NCH // 4, xform, 0)

        def count_ge(trial):
            def cnt(c, acc):
                for u4 in range(4):
                    k = key_v[pl.ds(c * 64 + u4 * 16, 16)]
                    acc = acc + jnp.where(k >= trial, 1, 0)
                return acc
            acc = lax.fori_loop(0, _NCH // 4, cnt, jnp.zeros((16,), jnp.int32))
            return _tree_sum(acc)

        def count_ge3(t01, t10, t11):
            def cnt(c, accs):
                a1, a2, a3 = accs
                for u4 in range(4):
                    k = key_v[pl.ds(c * 64 + u4 * 16, 16)]
                    a1 = a1 + jnp.where(k >= t01, 1, 0)
                    a2 = a2 + jnp.where(k >= t10, 1, 0)
                    a3 = a3 + jnp.where(k >= t11, 1, 0)
                return a1, a2, a3
            z = jnp.zeros((16,), jnp.int32)
            a1, a2, a3 = lax.fori_loop(0, _NCH // 4, cnt, (z, z, z))
            return _tree_sum(a1), _tree_sum(a2), _tree_sum(a3)

        P = jnp.zeros((16,), jnp.uint32)
        for b in range(31, -1, -2):
            hi = one_u << b
            lo = one_u << (b - 1)
            c01, c10, c11 = count_ge3(P | lo, P | hi, P | hi | lo)
            P = jnp.where(c01 >= k128, P | lo, P)
            P = jnp.where(c10 >= k128, P | hi, P)
            P = jnp.where(c11 >= k128, P | hi | lo, P)

        n_gt = count_ge(P + 1)
        tie_on = P != jnp.uint32(_MINF_UKEY)
        rt = jnp.where(tie_on, k128 - n_gt, 0)

        def emit(c, run):
            for u4 in range(4):
                k = key_v[pl.ds(c * 64 + u4 * 16, 16)]
                gt = k > P
                eq = k == P
                e = jnp.where(eq, 1, 0)
                pi = _prefix_incl(e)
                rank = run + pi - e
                sel = gt | (eq & (rank < rt))
                out_v[j, pl.ds(c * 64 + u4 * 16, 16)] = jnp.where(sel, 0.0, NEG_INF)
                run = run + jnp.take(pi, jnp.full((16,), 15, jnp.int32))
            return run
        lax.fori_loop(0, _NCH // 4, emit, jnp.zeros((16,), jnp.int32))
        return carry

    lax.fori_loop(0, _RPW // 8, process_group, 0)


def _sc_topk_mask(scores):
    return pl.kernel(
        _sc_topk_body,
        out_type=jax.ShapeDtypeStruct((T, T), jnp.float32),
        mesh=plsc.VectorSubcoreMesh(core_axis_name="c", subcore_axis_name="s"),
        scratch_types=[
            pltpu.VMEM((8, T), jnp.float32),
            pltpu.VMEM((8, T), jnp.float32),
            pltpu.VMEM((T,), jnp.uint32),
        ],
    )(scores)


def kernel(hidden_states, cos, sin, ks, ke, Wqa, q_a_ln_w, Wqb, Wkva, kv_a_ln_w,
           Wkvb, Wo, Wiq, Wik, ik_ln_w, ik_ln_b, Wiw):
    f32 = jnp.float32
    x = hidden_states[0]

    cs = cos[0, :, :ROPE // 2]
    sn = sin[0, :, :ROPE // 2]
    cos_il = jnp.repeat(cs, 2, axis=1)
    sin_sgn = jnp.stack([-sn, sn], axis=-1).reshape(T, ROPE)

    ks32 = jnp.broadcast_to(ks[:, None], (T, 128)).astype(jnp.int32)
    ke32 = jnp.broadcast_to(ke[:, None], (T, 128)).astype(jnp.int32)

    TB1 = 256
    qlat, kv, kvT, kidxT, wind = pl.pallas_call(
        _k1_body,
        grid=(T // TB1,),
        in_specs=[
            pl.BlockSpec((TB1, HID), lambda i: (i, 0)),
            _const_spec((QLORA, HID)),
            _const_spec((KVLORA + ROPE, HID)),
            _const_spec((IHD, HID)),
            _const_spec((IH, HID)),
            _const_spec((1, QLORA)),
            _const_spec((1, KVLORA)),
            _const_spec((1, IHD)),
            _const_spec((1, IHD)),
            pl.BlockSpec((TB1, ROPE), lambda i: (i, 0)),
            pl.BlockSpec((TB1, ROPE), lambda i: (i, 0)),
        ],
        out_specs=[
            pl.BlockSpec((TB1, QLORA), lambda i: (i, 0)),
            pl.BlockSpec((TB1, KVLORA + ROPE), lambda i: (i, 0)),
            pl.BlockSpec((KVLORA + ROPE, TB1), lambda i: (0, i)),
            pl.BlockSpec((IHD, TB1), lambda i: (0, i)),
            pl.BlockSpec((TB1, IH), lambda i: (i, 0)),
        ],
        out_shape=[
            jax.ShapeDtypeStruct((T, QLORA), f32),
            jax.ShapeDtypeStruct((T, KVLORA + ROPE), f32),
            jax.ShapeDtypeStruct((KVLORA + ROPE, T), f32),
            jax.ShapeDtypeStruct((IHD, T), f32),
            jax.ShapeDtypeStruct((T, IH), f32),
        ],
    )(x, Wqa, Wkva, Wik, Wiw, q_a_ln_w[None], kv_a_ln_w[None],
      ik_ln_w[None], ik_ln_b[None], cos_il, sin_sgn)

    TB2 = 256
    qi = pl.pallas_call(
        _k2a_body,
        grid=(T // TB2,),
        in_specs=[
            pl.BlockSpec((TB2, QLORA), lambda i: (i, 0)),
            _const_spec((IH * IHD, QLORA)),
            pl.BlockSpec((TB2, ROPE), lambda i: (i, 0)),
            pl.BlockSpec((TB2, ROPE), lambda i: (i, 0)),
        ],
        out_specs=pl.BlockSpec((TB2, IH * IHD), lambda i: (i, 0)),
        out_shape=jax.ShapeDtypeStruct((T, IH * IHD), f32),
    )(qlat, Wiq, cos_il, sin_sgn)

    TB3 = 256
    scores = pl.pallas_call(
        _k3_body,
        grid=(T // TB3,),
        in_specs=[
            pl.BlockSpec((TB3, IH * IHD), lambda i: (i, 0)),
            _const_spec((IHD, T)),
            pl.BlockSpec((TB3, IH), lambda i: (i, 0)),
            pl.BlockSpec((TB3, 128), lambda i: (i, 0)),
            pl.BlockSpec((TB3, 128), lambda i: (i, 0)),
        ],
        out_specs=pl.BlockSpec((TB3, T), lambda i: (i, 0)),
        out_shape=jax.ShapeDtypeStruct((T, T), f32),
    )(qi, kidxT, wind, ks32, ke32)

    maskbias = _sc_topk_mask(scores)

    TB2B = 128
    sq = pl.pallas_call(
        _k2b_body,
        grid=(T // TB2B,),
        in_specs=[
            pl.BlockSpec((TB2B, QLORA), lambda i: (i, 0)),
            _const_spec((H * QKHD, QLORA)),
            _const_spec((H * (NOPE + VHD), KVLORA)),
            pl.BlockSpec((TB2B, ROPE), lambda i: (i, 0)),
            pl.BlockSpec((TB2B, ROPE), lambda i: (i, 0)),
        ],
        out_specs=pl.BlockSpec((H, TB2B, KVLORA + ROPE), lambda i: (0, i, 0)),
        out_shape=jax.ShapeDtypeStruct((H, T, KVLORA + ROPE), f32),
    )(qlat, Wqb, Wkvb, cos_il, sin_sgn)

    TB4 = 256
    ao = pl.pallas_call(
        _k4_body,
        grid=(T // TB4, H),
        in_specs=[
            pl.BlockSpec((1, TB4, KVLORA + ROPE), lambda i, j: (j, i, 0)),
            _const_spec((KVLORA + ROPE, T)),
            _const_spec((T, KVLORA + ROPE)),
            pl.BlockSpec((TB4, T), lambda i, j: (i, 0)),
        ],
        out_specs=pl.BlockSpec((1, TB4, KVLORA), lambda i, j: (j, i, 0)),
        out_shape=jax.ShapeDtypeStruct((H, T, KVLORA), f32),
    )(sq, kvT, kv, maskbias)

    TB5 = 256
    out = pl.pallas_call(
        _k5_body,
        grid=(T // TB5,),
        in_specs=[
            pl.BlockSpec((H, TB5, KVLORA), lambda i: (0, i, 0)),
            _const_spec((H * (NOPE + VHD), KVLORA)),
            _const_spec((HID, H * VHD)),
        ],
        out_specs=pl.BlockSpec((TB5, HID), lambda i: (i, 0)),
        out_shape=jax.ShapeDtypeStruct((T, HID), f32),
    )(ao, Wkvb, Wo)

    return out[None]


# speedup vs baseline: 1.1623x; 1.1623x over previous
import functools

import jax
import jax.numpy as jnp
from jax import lax
from jax.experimental import pallas as pl
from jax.experimental.pallas import tpu as pltpu
from jax.experimental.pallas import tpu_sc as plsc

T = 2048; HID = 2048; H = 16; QLORA = 1536; KVLORA = 512
ROPE = 64; NOPE = 128; QKHD = 192; VHD = 128
IH = 16; IHD = 128; TOPK = 128
SCALE = QKHD ** -0.5
WSCALE = (IHD ** -0.5) * (IH ** -0.5)
NEG_INF = float("-inf")

HIGH = "bf16"
DEFAULT = "bf16"

C_QL = 0
C_KC = 1536
C_KI = 2048
C_KR = 2176
C_W = 2304
K1_COLS = 2432

K2_COLS = 5120


def _dot(a, b, prec):
    if prec == "bf16":
        a = a.astype(jnp.bfloat16)
        b = b.astype(jnp.bfloat16)
        prec = None
    return jax.lax.dot_general(a, b, (((1,), (0,)), ((), ())),
                               precision=prec, preferred_element_type=jnp.float32)


def _dot_bT(a, b, prec):
    if prec == "bf16":
        a = a.astype(jnp.bfloat16)
        b = b.astype(jnp.bfloat16)
        prec = None
    return jax.lax.dot_general(a, b, (((1,), (1,)), ((), ())),
                               precision=prec, preferred_element_type=jnp.float32)


def _rms(x, w, eps=1e-6):
    return x * jax.lax.rsqrt(jnp.mean(x * x, -1, keepdims=True) + eps) * w


def _ln(x, w, b, eps=1e-6):
    mu = jnp.mean(x, -1, keepdims=True)
    var = jnp.mean((x - mu) ** 2, -1, keepdims=True)
    return (x - mu) * jax.lax.rsqrt(var + eps) * w + b


def _swap_pairs(x):
    even = jax.lax.broadcasted_iota(jnp.int32, x.shape, x.ndim - 1) % 2 == 0
    return jnp.where(even, jnp.roll(x, -1, axis=-1), jnp.roll(x, 1, axis=-1))


def _rope_il(x, cos_il, sin_sgn):
    return x * cos_il + _swap_pairs(x) * sin_sgn


def _k1_body(x_ref, wqa_ref, wkva_ref, wik_ref, wiw_ref,
             qaln_ref, kvln_ref, ikw_ref, ikb_ref, cos_ref, sin_ref,
             qlat_ref, kv_ref, kvT_ref, kidxT_ref, wind_ref):
    x = x_ref[...]
    qlat_ref[...] = _rms(_dot_bT(x, wqa_ref[...], HIGH), qaln_ref[...])
    ckv = _dot_bT(x, wkva_ref[...], HIGH)
    k_comp = _rms(ckv[:, :KVLORA], kvln_ref[...])
    cos = cos_ref[...]; sin = sin_ref[...]
    k_rope = _rope_il(ckv[:, KVLORA:], cos, sin)
    kv = jnp.concatenate([k_comp, k_rope], axis=1)
    kv_ref[...] = kv
    kvT_ref[...] = kv.T
    ki = _ln(_dot_bT(x, wik_ref[...], HIGH), ikw_ref[...], ikb_ref[...])
    tb = ki.shape[0]
    ccat = jnp.concatenate([cos, jnp.ones((tb, IHD - ROPE), jnp.float32)], axis=1)
    scat = jnp.concatenate([sin, jnp.zeros((tb, IHD - ROPE), jnp.float32)], axis=1)
    kidxT_ref[...] = (ki * ccat + _swap_pairs(ki) * scat).T
    wind_ref[...] = _dot_bT(x, wiw_ref[...], HIGH)


def _k2a_body(qlat_ref, wiq_ref, cos_ref, sin_ref, qi_ref):
    ql = qlat_ref[...]
    cos = cos_ref[...]; sin = sin_ref[...]
    tb = ql.shape[0]
    ccat = jnp.concatenate([cos, jnp.ones((tb, IHD - ROPE), jnp.float32)], axis=1)
    scat = jnp.concatenate([sin, jnp.zeros((tb, IHD - ROPE), jnp.float32)], axis=1)
    for h in range(IH):
        qh = _dot_bT(ql, wiq_ref[h * IHD:(h + 1) * IHD], HIGH)
        qi_ref[:, h * IHD:(h + 1) * IHD] = qh * ccat + _swap_pairs(qh) * scat


def _k2b_body(qlat_ref, wqb_ref, wkvb_ref, cos_ref, sin_ref, sq_ref):
    ql = qlat_ref[...]
    cos = cos_ref[...]; sin = sin_ref[...]
    for h in range(H):
        qf = _dot_bT(ql, wqb_ref[h * QKHD:(h + 1) * QKHD], HIGH)
        qa = _dot(qf[:, :NOPE],
                  wkvb_ref[h * (NOPE + VHD):h * (NOPE + VHD) + NOPE], HIGH)
        qr = _rope_il(qf[:, NOPE:], cos, sin)
        sq_ref[h] = jnp.concatenate([qa, qr], axis=1)


def _k3_body(qi_ref, kT_ref, w_ref, ks_ref, ke_ref, out_ref):
    tb = qi_ref.shape[0]
    acc = jnp.zeros((tb, T), jnp.float32)
    w = w_ref[...]
    for h in range(IH):
        s = _dot(qi_ref[:, h * IHD:(h + 1) * IHD], kT_ref[...], HIGH)
        acc = acc + jnp.maximum(s, 0.0) * w[:, h:h + 1]
    acc = acc * WSCALE
    pos = jax.lax.broadcasted_iota(jnp.int32, (tb, T), 1)
    valid = (pos >= ks_ref[:, :1]) & (pos < ke_ref[:, :1])
    out_ref[...] = jnp.where(valid, acc, NEG_INF)


def _k4_body(sq_ref, kvT_ref, kv_ref, mask_ref, out_ref):
    q = sq_ref[0]
    l = _dot(q, kvT_ref[...], DEFAULT) * SCALE + mask_ref[...]
    m = jnp.max(l, axis=-1, keepdims=True)
    p = jnp.exp(l - m)
    p = p / jnp.sum(p, axis=-1, keepdims=True)
    out_ref[0] = _dot(p, kv_ref[:, :KVLORA], DEFAULT)


def _k5_body(ao_ref, wkvb_ref, wo_ref, out_ref):
    cat = jnp.concatenate(
        [_dot_bT(ao_ref[h],
                 wkvb_ref[h * (NOPE + VHD) + NOPE:(h + 1) * (NOPE + VHD)],
                 DEFAULT) for h in range(H)], axis=1)
    out_ref[...] = _dot_bT(cat, wo_ref[...], DEFAULT)


def _const_spec(shape):
    return pl.BlockSpec(shape, lambda *args: tuple(0 for _ in shape))


_NW = 32
_RPW = T // _NW
_NCH = T // 16
_MINF_UKEY = 0x007FFFFF


def _lanes():
    return lax.iota(jnp.int32, 16)


def _tree_sum(v):
    ln = _lanes()
    for sh in (1, 2, 4, 8):
        v = v + jnp.take(v, ln ^ sh)
    return v


def _prefix_incl(v):
    ln = _lanes()
    for sh in (1, 2, 4, 8):
        t = jnp.take(v, jnp.maximum(ln - sh, 0))
        v = v + jnp.where(ln >= sh, t, 0)
    return v


def _sc_topk_body(scores_hbm, mask_hbm, row_v, out_v, key_v):
    wid = lax.axis_index("s") * 2 + lax.axis_index("c")
    one_u = jnp.full((16,), 1, jnp.uint32)
    k128 = jnp.full((16,), TOPK, jnp.int32)

    def process_row(i, carry):
        r = wid + _NW * i
        pltpu.sync_copy(scores_hbm.at[r], row_v)

        def xform(c, _):
            for u4 in range(4):
                x = row_v[pl.ds(c * 64 + u4 * 16, 16)]
                u = lax.bitcast_convert_type(x, jnp.uint32)
                flip = jnp.where(u >> 31 != 0,
                                 jnp.uint32(0xFFFFFFFF), jnp.uint32(0x80000000))
                key_v[pl.ds(c * 64 + u4 * 16, 16)] = u ^ flip
            return 0
        lax.fori_loop(0, _NCH // 4, xform, 0)

        def count_ge(trial):
            def cnt(c, acc):
                for u4 in range(4):
                    k = key_v[pl.ds(c * 64 + u4 * 16, 16)]
                    acc = acc + jnp.where(k >= trial, 1, 0)
                return acc
            acc = lax.fori_loop(0, _NCH // 4, cnt, jnp.zeros((16,), jnp.int32))
            return _tree_sum(acc)

        def count_ge3(t01, t10, t11):
            def cnt(c, accs):
                a1, a2, a3 = accs
                for u4 in range(4):
                    k = key_v[pl.ds(c * 64 + u4 * 16, 16)]
                    a1 = a1 + jnp.where(k >= t01, 1, 0)
                    a2 = a2 + jnp.where(k >= t10, 1, 0)
                    a3 = a3 + jnp.where(k >= t11, 1, 0)
                return a1, a2, a3
            z = jnp.zeros((16,), jnp.int32)
            a1, a2, a3 = lax.fori_loop(0, _NCH // 4, cnt, (z, z, z))
            return _tree_sum(a1), _tree_sum(a2), _tree_sum(a3)

        P = jnp.zeros((16,), jnp.uint32)
        for b in range(31, -1, -2):
            hi = one_u << b
            lo = one_u << (b - 1)
            c01, c10, c11 = count_ge3(P | lo, P | hi, P | hi | lo)
            P = jnp.where(c01 >= k128, P | lo, P)
            P = jnp.where(c10 >= k128, P | hi, P)
            P = jnp.where(c11 >= k128, P | hi | lo, P)

        n_gt = count_ge(P + 1)
        tie_on = P != jnp.uint32(_MINF_UKEY)
        rt = jnp.where(tie_on, k128 - n_gt, 0)

        def emit(c, run):
            for u4 in range(4):
                k = key_v[pl.ds(c * 64 + u4 * 16, 16)]
                gt = k > P
                eq = k == P
                e = jnp.where(eq, 1, 0)
                pi = _prefix_incl(e)
                rank = run + pi - e
                sel = gt | (eq & (rank < rt))
                out_v[pl.ds(c * 64 + u4 * 16, 16)] = jnp.where(sel, 0.0, NEG_INF)
                run = run + jnp.take(pi, jnp.full((16,), 15, jnp.int32))
            return run
        lax.fori_loop(0, _NCH // 4, emit, jnp.zeros((16,), jnp.int32))

        pltpu.sync_copy(out_v, mask_hbm.at[r])
        return carry

    lax.fori_loop(0, _RPW, process_row, 0)


def _sc_topk_mask(scores):
    return pl.kernel(
        _sc_topk_body,
        out_type=jax.ShapeDtypeStruct((T, T), jnp.float32),
        mesh=plsc.VectorSubcoreMesh(core_axis_name="c", subcore_axis_name="s"),
        scratch_types=[
            pltpu.VMEM((T,), jnp.float32),
            pltpu.VMEM((T,), jnp.float32),
            pltpu.VMEM((T,), jnp.uint32),
        ],
    )(scores)


def kernel(hidden_states, cos, sin, ks, ke, Wqa, q_a_ln_w, Wqb, Wkva, kv_a_ln_w,
           Wkvb, Wo, Wiq, Wik, ik_ln_w, ik_ln_b, Wiw):
    f32 = jnp.float32
    x = hidden_states[0]

    cs = cos[0, :, :ROPE // 2]
    sn = sin[0, :, :ROPE // 2]
    cos_il = jnp.repeat(cs, 2, axis=1)
    sin_sgn = jnp.stack([-sn, sn], axis=-1).reshape(T, ROPE)

    ks32 = jnp.broadcast_to(ks[:, None], (T, 128)).astype(jnp.int32)
    ke32 = jnp.broadcast_to(ke[:, None], (T, 128)).astype(jnp.int32)

    TB1 = 256
    qlat, kv, kvT, kidxT, wind = pl.pallas_call(
        _k1_body,
        grid=(T // TB1,),
        in_specs=[
            pl.BlockSpec((TB1, HID), lambda i: (i, 0)),
            _const_spec((QLORA, HID)),
            _const_spec((KVLORA + ROPE, HID)),
            _const_spec((IHD, HID)),
            _const_spec((IH, HID)),
            _const_spec((1, QLORA)),
            _const_spec((1, KVLORA)),
            _const_spec((1, IHD)),
            _const_spec((1, IHD)),
            pl.BlockSpec((TB1, ROPE), lambda i: (i, 0)),
            pl.BlockSpec((TB1, ROPE), lambda i: (i, 0)),
        ],
        out_specs=[
            pl.BlockSpec((TB1, QLORA), lambda i: (i, 0)),
            pl.BlockSpec((TB1, KVLORA + ROPE), lambda i: (i, 0)),
            pl.BlockSpec((KVLORA + ROPE, TB1), lambda i: (0, i)),
            pl.BlockSpec((IHD, TB1), lambda i: (0, i)),
            pl.BlockSpec((TB1, IH), lambda i: (i, 0)),
        ],
        out_shape=[
            jax.ShapeDtypeStruct((T, QLORA), f32),
            jax.ShapeDtypeStruct((T, KVLORA + ROPE), f32),
            jax.ShapeDtypeStruct((KVLORA + ROPE, T), f32),
            jax.ShapeDtypeStruct((IHD, T), f32),
            jax.ShapeDtypeStruct((T, IH), f32),
        ],
    )(x, Wqa, Wkva, Wik, Wiw, q_a_ln_w[None], kv_a_ln_w[None],
      ik_ln_w[None], ik_ln_b[None], cos_il, sin_sgn)

    TB2 = 256
    qi = pl.pallas_call(
        _k2a_body,
        grid=(T // TB2,),
        in_specs=[
            pl.BlockSpec((TB2, QLORA), lambda i: (i, 0)),
            _const_spec((IH * IHD, QLORA)),
            pl.BlockSpec((TB2, ROPE), lambda i: (i, 0)),
            pl.BlockSpec((TB2, ROPE), lambda i: (i, 0)),
        ],
        out_specs=pl.BlockSpec((TB2, IH * IHD), lambda i: (i, 0)),
        out_shape=jax.ShapeDtypeStruct((T, IH * IHD), f32),
    )(qlat, Wiq, cos_il, sin_sgn)

    TB3 = 256
    scores = pl.pallas_call(
        _k3_body,
        grid=(T // TB3,),
        in_specs=[
            pl.BlockSpec((TB3, IH * IHD), lambda i: (i, 0)),
            _const_spec((IHD, T)),
            pl.BlockSpec((TB3, IH), lambda i: (i, 0)),
            pl.BlockSpec((TB3, 128), lambda i: (i, 0)),
            pl.BlockSpec((TB3, 128), lambda i: (i, 0)),
        ],
        out_specs=pl.BlockSpec((TB3, T), lambda i: (i, 0)),
        out_shape=jax.ShapeDtypeStruct((T, T), f32),
    )(qi, kidxT, wind, ks32, ke32)

    maskbias = _sc_topk_mask(scores)

    TB2B = 256
    sq = pl.pallas_call(
        _k2b_body,
        grid=(T // TB2B,),
        in_specs=[
            pl.BlockSpec((TB2B, QLORA), lambda i: (i, 0)),
            _const_spec((H * QKHD, QLORA)),
            _const_spec((H * (NOPE + VHD), KVLORA)),
            pl.BlockSpec((TB2B, ROPE), lambda i: (i, 0)),
            pl.BlockSpec((TB2B, ROPE), lambda i: (i, 0)),
        ],
        out_specs=pl.BlockSpec((H, TB2B, KVLORA + ROPE), lambda i: (0, i, 0)),
        out_shape=jax.ShapeDtypeStruct((H, T, KVLORA + ROPE), f32),
    )(qlat, Wqb, Wkvb, cos_il, sin_sgn)

    TB4 = 512
    ao = pl.pallas_call(
        _k4_body,
        grid=(T // TB4, H),
        in_specs=[
            pl.BlockSpec((1, TB4, KVLORA + ROPE), lambda i, j: (j, i, 0)),
            _const_spec((KVLORA + ROPE, T)),
            _const_spec((T, KVLORA + ROPE)),
            pl.BlockSpec((TB4, T), lambda i, j: (i, 0)),
        ],
        out_specs=pl.BlockSpec((1, TB4, KVLORA), lambda i, j: (j, i, 0)),
        out_shape=jax.ShapeDtypeStruct((H, T, KVLORA), f32),
    )(sq, kvT, kv, maskbias)

    TB5 = 256
    out = pl.pallas_call(
        _k5_body,
        grid=(T // TB5,),
        in_specs=[
            pl.BlockSpec((H, TB5, KVLORA), lambda i: (0, i, 0)),
            _const_spec((H * (NOPE + VHD), KVLORA)),
            _const_spec((HID, H * VHD)),
        ],
        out_specs=pl.BlockSpec((TB5, HID), lambda i: (i, 0)),
        out_shape=jax.ShapeDtypeStruct((T, HID), f32),
    )(ao, Wkvb, Wo)

    return out[None]


# speedup vs baseline: 1.1727x; 1.0089x over previous
import functools

import jax
import jax.numpy as jnp
from jax import lax
from jax.experimental import pallas as pl
from jax.experimental.pallas import tpu as pltpu
from jax.experimental.pallas import tpu_sc as plsc

T = 2048; HID = 2048; H = 16; QLORA = 1536; KVLORA = 512
ROPE = 64; NOPE = 128; QKHD = 192; VHD = 128
IH = 16; IHD = 128; TOPK = 128
SCALE = QKHD ** -0.5
WSCALE = (IHD ** -0.5) * (IH ** -0.5)
NEG_INF = float("-inf")

HIGH = "bf16"
DEFAULT = "bf16"

C_QL = 0
C_KC = 1536
C_KI = 2048
C_KR = 2176
C_W = 2304
K1_COLS = 2432

K2_COLS = 5120


def _dot(a, b, prec):
    if prec == "bf16":
        a = a.astype(jnp.bfloat16)
        b = b.astype(jnp.bfloat16)
        prec = None
    return jax.lax.dot_general(a, b, (((1,), (0,)), ((), ())),
                               precision=prec, preferred_element_type=jnp.float32)


def _dot_bT(a, b, prec):
    if prec == "bf16":
        a = a.astype(jnp.bfloat16)
        b = b.astype(jnp.bfloat16)
        prec = None
    return jax.lax.dot_general(a, b, (((1,), (1,)), ((), ())),
                               precision=prec, preferred_element_type=jnp.float32)


def _rms(x, w, eps=1e-6):
    return x * jax.lax.rsqrt(jnp.mean(x * x, -1, keepdims=True) + eps) * w


def _ln(x, w, b, eps=1e-6):
    mu = jnp.mean(x, -1, keepdims=True)
    var = jnp.mean((x - mu) ** 2, -1, keepdims=True)
    return (x - mu) * jax.lax.rsqrt(var + eps) * w + b


def _swap_pairs(x):
    even = jax.lax.broadcasted_iota(jnp.int32, x.shape, x.ndim - 1) % 2 == 0
    return jnp.where(even, jnp.roll(x, -1, axis=-1), jnp.roll(x, 1, axis=-1))


def _rope_il(x, cos_il, sin_sgn):
    return x * cos_il + _swap_pairs(x) * sin_sgn


def _k1_body(x_ref, wqa_ref, wkva_ref, wik_ref, wiw_ref,
             qaln_ref, kvln_ref, ikw_ref, ikb_ref, cos_ref, sin_ref,
             qlat_ref, kv_ref, kvT_ref, kidxT_ref, wind_ref):
    x = x_ref[...]
    qlat_ref[...] = _rms(_dot_bT(x, wqa_ref[...], HIGH), qaln_ref[...])
    ckv = _dot_bT(x, wkva_ref[...], HIGH)
    k_comp = _rms(ckv[:, :KVLORA], kvln_ref[...])
    cos = cos_ref[...]; sin = sin_ref[...]
    k_rope = _rope_il(ckv[:, KVLORA:], cos, sin)
    kv = jnp.concatenate([k_comp, k_rope], axis=1)
    kv_ref[...] = kv
    kvT_ref[...] = kv.T
    ki = _ln(_dot_bT(x, wik_ref[...], HIGH), ikw_ref[...], ikb_ref[...])
    tb = ki.shape[0]
    ccat = jnp.concatenate([cos, jnp.ones((tb, IHD - ROPE), jnp.float32)], axis=1)
    scat = jnp.concatenate([sin, jnp.zeros((tb, IHD - ROPE), jnp.float32)], axis=1)
    kidxT_ref[...] = (ki * ccat + _swap_pairs(ki) * scat).T
    wind_ref[...] = _dot_bT(x, wiw_ref[...], HIGH)


def _k2a_body(qlat_ref, wiq_ref, cos_ref, sin_ref, qi_ref):
    ql = qlat_ref[...]
    cos = cos_ref[...]; sin = sin_ref[...]
    tb = ql.shape[0]
    ccat = jnp.concatenate([cos, jnp.ones((tb, IHD - ROPE), jnp.float32)], axis=1)
    scat = jnp.concatenate([sin, jnp.zeros((tb, IHD - ROPE), jnp.float32)], axis=1)
    for h in range(IH):
        qh = _dot_bT(ql, wiq_ref[h * IHD:(h + 1) * IHD], HIGH)
        qi_ref[:, h * IHD:(h + 1) * IHD] = qh * ccat + _swap_pairs(qh) * scat


def _k2b_body(qlat_ref, wqb_ref, wkvb_ref, cos_ref, sin_ref, sq_ref):
    ql = qlat_ref[...]
    cos = cos_ref[...]; sin = sin_ref[...]
    for h in range(H):
        qf = _dot_bT(ql, wqb_ref[h * QKHD:(h + 1) * QKHD], HIGH)
        qa = _dot(qf[:, :NOPE],
                  wkvb_ref[h * (NOPE + VHD):h * (NOPE + VHD) + NOPE], HIGH)
        qr = _rope_il(qf[:, NOPE:], cos, sin)
        sq_ref[h] = jnp.concatenate([qa, qr], axis=1)


def _k3_body(qi_ref, kT_ref, w_ref, ks_ref, ke_ref, out_ref):
    tb = qi_ref.shape[0]
    acc = jnp.zeros((tb, T), jnp.float32)
    w = w_ref[...]
    for h in range(IH):
        s = _dot(qi_ref[:, h * IHD:(h + 1) * IHD], kT_ref[...], HIGH)
        acc = acc + jnp.maximum(s, 0.0) * w[:, h:h + 1]
    acc = acc * WSCALE
    pos = jax.lax.broadcasted_iota(jnp.int32, (tb, T), 1)
    valid = (pos >= ks_ref[:, :1]) & (pos < ke_ref[:, :1])
    out_ref[...] = jnp.where(valid, acc, NEG_INF)


def _k4_body(sq_ref, kvT_ref, kv_ref, mask_ref, out_ref):
    q = sq_ref[0]
    l = _dot(q, kvT_ref[...], DEFAULT) * SCALE + mask_ref[...]
    m = jnp.max(l, axis=-1, keepdims=True)
    p = jnp.exp(l - m)
    p = p / jnp.sum(p, axis=-1, keepdims=True)
    out_ref[0] = _dot(p, kv_ref[:, :KVLORA], DEFAULT)


def _k5_body(ao_ref, wkvb_ref, wo_ref, out_ref):
    cat = jnp.concatenate(
        [_dot_bT(ao_ref[h],
                 wkvb_ref[h * (NOPE + VHD) + NOPE:(h + 1) * (NOPE + VHD)],
                 DEFAULT) for h in range(H)], axis=1)
    out_ref[...] = _dot_bT(cat, wo_ref[...], DEFAULT)


def _const_spec(shape):
    return pl.BlockSpec(shape, lambda *args: tuple(0 for _ in shape))


_NW = 32
_RPW = T // _NW
_NCH = T // 16
_MINF_UKEY = 0x007FFFFF


def _lanes():
    return lax.iota(jnp.int32, 16)


def _tree_sum(v):
    ln = _lanes()
    for sh in (1, 2, 4, 8):
        v = v + jnp.take(v, ln ^ sh)
    return v


def _prefix_incl(v):
    ln = _lanes()
    for sh in (1, 2, 4, 8):
        t = jnp.take(v, jnp.maximum(ln - sh, 0))
        v = v + jnp.where(ln >= sh, t, 0)
    return v


def _sc_topk_body(scores_hbm, mask_hbm, row_v, out_v, key_v):
    wid = lax.axis_index("s") * 2 + lax.axis_index("c")
    one_u = jnp.full((16,), 1, jnp.uint32)
    k128 = jnp.full((16,), TOPK, jnp.int32)

    def process_row(i, carry):
        r = wid + _NW * i
        pltpu.sync_copy(scores_hbm.at[r], row_v)

        def xform(c, _):
            for u4 in range(4):
                x = row_v[pl.ds(c * 64 + u4 * 16, 16)]
                u = lax.bitcast_convert_type(x, jnp.uint32)
                flip = jnp.where(u >> 31 != 0,
                                 jnp.uint32(0xFFFFFFFF), jnp.uint32(0x80000000))
                key_v[pl.ds(c * 64 + u4 * 16, 16)] = u ^ flip
            return 0
        lax.fori_loop(0, _NCH // 4, xform, 0)

        def count_ge(trial):
            def cnt(c, acc):
                for u4 in range(4):
                    k = key_v[pl.ds(c * 64 + u4 * 16, 16)]
                    acc = acc + jnp.where(k >= trial, 1, 0)
                return acc
            acc = lax.fori_loop(0, _NCH // 4, cnt, jnp.zeros((16,), jnp.int32))
            return _tree_sum(acc)

        def count_ge3(t01, t10, t11):
            def cnt(c, accs):
                a1, a2, a3 = accs
                for u4 in range(4):
                    k = key_v[pl.ds(c * 64 + u4 * 16, 16)]
                    a1 = a1 + jnp.where(k >= t01, 1, 0)
                    a2 = a2 + jnp.where(k >= t10, 1, 0)
                    a3 = a3 + jnp.where(k >= t11, 1, 0)
                return a1, a2, a3
            z = jnp.zeros((16,), jnp.int32)
            a1, a2, a3 = lax.fori_loop(0, _NCH // 4, cnt, (z, z, z))
            return _tree_sum(a1), _tree_sum(a2), _tree_sum(a3)

        P = jnp.zeros((16,), jnp.uint32)
        for b in range(31, -1, -2):
            hi = one_u << b
            lo = one_u << (b - 1)
            c01, c10, c11 = count_ge3(P | lo, P | hi, P | hi | lo)
            P = jnp.where(c01 >= k128, P | lo, P)
            P = jnp.where(c10 >= k128, P | hi, P)
            P = jnp.where(c11 >= k128, P | hi | lo, P)

        n_gt = count_ge(P + 1)
        tie_on = P != jnp.uint32(_MINF_UKEY)
        rt = jnp.where(tie_on, k128 - n_gt, 0)

        def emit(c, run):
            for u4 in range(4):
                k = key_v[pl.ds(c * 64 + u4 * 16, 16)]
                gt = k > P
                eq = k == P
                e = jnp.where(eq, 1, 0)
                pi = _prefix_incl(e)
                rank = run + pi - e
                sel = gt | (eq & (rank < rt))
                out_v[pl.ds(c * 64 + u4 * 16, 16)] = jnp.where(sel, 0.0, NEG_INF)
                run = run + jnp.take(pi, jnp.full((16,), 15, jnp.int32))
            return run
        lax.fori_loop(0, _NCH // 4, emit, jnp.zeros((16,), jnp.int32))

        pltpu.sync_copy(out_v, mask_hbm.at[r])
        return carry

    lax.fori_loop(0, _RPW, process_row, 0)


def _sc_topk_mask(scores):
    return pl.kernel(
        _sc_topk_body,
        out_type=jax.ShapeDtypeStruct((T, T), jnp.float32),
        mesh=plsc.VectorSubcoreMesh(core_axis_name="c", subcore_axis_name="s"),
        scratch_types=[
            pltpu.VMEM((T,), jnp.float32),
            pltpu.VMEM((T,), jnp.float32),
            pltpu.VMEM((T,), jnp.uint32),
        ],
    )(scores)


def kernel(hidden_states, cos, sin, ks, ke, Wqa, q_a_ln_w, Wqb, Wkva, kv_a_ln_w,
           Wkvb, Wo, Wiq, Wik, ik_ln_w, ik_ln_b, Wiw):
    f32 = jnp.float32
    x = hidden_states[0]

    cs = cos[0, :, :ROPE // 2]
    sn = sin[0, :, :ROPE // 2]
    cos_il = jnp.repeat(cs, 2, axis=1)
    sin_sgn = jnp.stack([-sn, sn], axis=-1).reshape(T, ROPE)

    ks32 = jnp.broadcast_to(ks[:, None], (T, 128)).astype(jnp.int32)
    ke32 = jnp.broadcast_to(ke[:, None], (T, 128)).astype(jnp.int32)

    TB1 = 512
    qlat, kv, kvT, kidxT, wind = pl.pallas_call(
        _k1_body,
        grid=(T // TB1,),
        in_specs=[
            pl.BlockSpec((TB1, HID), lambda i: (i, 0)),
            _const_spec((QLORA, HID)),
            _const_spec((KVLORA + ROPE, HID)),
            _const_spec((IHD, HID)),
            _const_spec((IH, HID)),
            _const_spec((1, QLORA)),
            _const_spec((1, KVLORA)),
            _const_spec((1, IHD)),
            _const_spec((1, IHD)),
            pl.BlockSpec((TB1, ROPE), lambda i: (i, 0)),
            pl.BlockSpec((TB1, ROPE), lambda i: (i, 0)),
        ],
        out_specs=[
            pl.BlockSpec((TB1, QLORA), lambda i: (i, 0)),
            pl.BlockSpec((TB1, KVLORA + ROPE), lambda i: (i, 0)),
            pl.BlockSpec((KVLORA + ROPE, TB1), lambda i: (0, i)),
            pl.BlockSpec((IHD, TB1), lambda i: (0, i)),
            pl.BlockSpec((TB1, IH), lambda i: (i, 0)),
        ],
        out_shape=[
            jax.ShapeDtypeStruct((T, QLORA), f32),
            jax.ShapeDtypeStruct((T, KVLORA + ROPE), f32),
            jax.ShapeDtypeStruct((KVLORA + ROPE, T), f32),
            jax.ShapeDtypeStruct((IHD, T), f32),
            jax.ShapeDtypeStruct((T, IH), f32),
        ],
    )(x, Wqa, Wkva, Wik, Wiw, q_a_ln_w[None], kv_a_ln_w[None],
      ik_ln_w[None], ik_ln_b[None], cos_il, sin_sgn)

    TB2 = 256
    qi = pl.pallas_call(
        _k2a_body,
        grid=(T // TB2,),
        in_specs=[
            pl.BlockSpec((TB2, QLORA), lambda i: (i, 0)),
            _const_spec((IH * IHD, QLORA)),
            pl.BlockSpec((TB2, ROPE), lambda i: (i, 0)),
            pl.BlockSpec((TB2, ROPE), lambda i: (i, 0)),
        ],
        out_specs=pl.BlockSpec((TB2, IH * IHD), lambda i: (i, 0)),
        out_shape=jax.ShapeDtypeStruct((T, IH * IHD), f32),
    )(qlat, Wiq, cos_il, sin_sgn)

    TB3 = 512
    scores = pl.pallas_call(
        _k3_body,
        grid=(T // TB3,),
        in_specs=[
            pl.BlockSpec((TB3, IH * IHD), lambda i: (i, 0)),
            _const_spec((IHD, T)),
            pl.BlockSpec((TB3, IH), lambda i: (i, 0)),
            pl.BlockSpec((TB3, 128), lambda i: (i, 0)),
            pl.BlockSpec((TB3, 128), lambda i: (i, 0)),
        ],
        out_specs=pl.BlockSpec((TB3, T), lambda i: (i, 0)),
        out_shape=jax.ShapeDtypeStruct((T, T), f32),
    )(qi, kidxT, wind, ks32, ke32)

    maskbias = _sc_topk_mask(scores)

    TB2B = 256
    sq = pl.pallas_call(
        _k2b_body,
        grid=(T // TB2B,),
        in_specs=[
            pl.BlockSpec((TB2B, QLORA), lambda i: (i, 0)),
            _const_spec((H * QKHD, QLORA)),
            _const_spec((H * (NOPE + VHD), KVLORA)),
            pl.BlockSpec((TB2B, ROPE), lambda i: (i, 0)),
            pl.BlockSpec((TB2B, ROPE), lambda i: (i, 0)),
        ],
        out_specs=pl.BlockSpec((H, TB2B, KVLORA + ROPE), lambda i: (0, i, 0)),
        out_shape=jax.ShapeDtypeStruct((H, T, KVLORA + ROPE), f32),
    )(qlat, Wqb, Wkvb, cos_il, sin_sgn)

    TB4 = 1024
    ao = pl.pallas_call(
        _k4_body,
        grid=(T // TB4, H),
        in_specs=[
            pl.BlockSpec((1, TB4, KVLORA + ROPE), lambda i, j: (j, i, 0)),
            _const_spec((KVLORA + ROPE, T)),
            _const_spec((T, KVLORA + ROPE)),
            pl.BlockSpec((TB4, T), lambda i, j: (i, 0)),
        ],
        out_specs=pl.BlockSpec((1, TB4, KVLORA), lambda i, j: (j, i, 0)),
        out_shape=jax.ShapeDtypeStruct((H, T, KVLORA), f32),
    )(sq, kvT, kv, maskbias)

    TB5 = 256
    out = pl.pallas_call(
        _k5_body,
        grid=(T // TB5,),
        in_specs=[
            pl.BlockSpec((H, TB5, KVLORA), lambda i: (0, i, 0)),
            _const_spec((H * (NOPE + VHD), KVLORA)),
            _const_spec((HID, H * VHD)),
        ],
        out_specs=pl.BlockSpec((TB5, HID), lambda i: (i, 0)),
        out_shape=jax.ShapeDtypeStruct((T, HID), f32),
    )(ao, Wkvb, Wo)

    return out[None]
